# baseline (speedup 1.0000x reference)
import jax
import jax.numpy as jnp
from jax import lax
from jax.experimental import pallas as pl
from jax.experimental.pallas import tpu as pltpu
from jax.experimental.pallas import tpu_sc as plsc

B, N, GRID = 16, 8192, 32768
NC, NS, L = 2, 16, 16
NW = NC * NS
HALF = N // 2
GB = GRID // 128
PB = HALF // 128


def _rsqrt(v):
    bits = lax.bitcast_convert_type(v, jnp.int32)
    r = lax.bitcast_convert_type(0x5F3759DF - (bits >> 1), jnp.float32)
    h = 0.5 * v
    r = r * (1.5 - h * r * r)
    r = r * (1.5 - h * r * r)
    r = r * (1.5 - h * r * r)
    return r


def _sc_body(grid_hbm, pts_hbm, prm_hbm, out_hbm, grid_v, pts_v, prm_v, cf_v, out_v, sem):
    wid = lax.axis_index("s") * NC + lax.axis_index("c")
    b = wid // 2
    half = wid % 2
    bhi = b // 8
    blo = b % 8

    cps = [pltpu.make_async_copy(prm_hbm, prm_v, sem)]
    for c in range(3):
        cps.append(pltpu.make_async_copy(
            grid_hbm.at[c, bhi, :, blo, :], grid_v.at[pl.ds(c * GB, GB), :], sem))
        cps.append(pltpu.make_async_copy(
            pts_hbm.at[c, bhi, pl.ds(half * PB, PB), blo, :],
            pts_v.at[pl.ds(c * PB, PB), :], sem))
    for cp in cps:
        cp.start()
    for cp in cps:
        cp.wait()

    for t in range(3):
        base = jnp.full((L,), b * 24 + 4 * t, jnp.int32)
        n0 = plsc.load_gather(prm_v, [base])
        n1 = plsc.load_gather(prm_v, [base + 1])
        n2 = plsc.load_gather(prm_v, [base + 2])
        d = plsc.load_gather(prm_v, [base + 3])
        inv = 1.0 / (n0 * n0 + n1 * n1 + n2 * n2)
        cf_v[5 * t, :] = n0
        cf_v[5 * t + 1, :] = n1
        cf_v[5 * t + 2, :] = n2
        cf_v[5 * t + 3, :] = d
        cf_v[5 * t + 4, :] = inv + inv
    for t in range(3):
        base = jnp.full((L,), b * 24 + 12 + 4 * t, jnp.int32)
        w = plsc.load_gather(prm_v, [base])
        x = plsc.load_gather(prm_v, [base + 1])
        y = plsc.load_gather(prm_v, [base + 2])
        z = plsc.load_gather(prm_v, [base + 3])
        rn = _rsqrt(w * w + x * x + y * y + z * z)
        ww, xx, yy, zz = w * w, x * x, y * y, z * z
        xy, xz, yz = x * y, x * z, y * z
        wx, wy, wz = w * x, w * y, w * z
        m = [
            (ww + xx - yy - zz) * rn, 2.0 * (xy - wz) * rn, 2.0 * (xz + wy) * rn,
            2.0 * (xy + wz) * rn, (ww - xx + yy - zz) * rn, 2.0 * (yz - wx) * rn,
            2.0 * (xz - wy) * rn, 2.0 * (yz + wx) * rn, (ww - xx - yy + zz) * rn,
        ]
        for k in range(9):
            cf_v[15 + 9 * t + k, :] = m[k]

    def _tail(sx, sy, sz, acc):
        ix = jnp.clip(sx, 0.0, 32.0).astype(jnp.int32)
        iy = jnp.clip(sy, 0.0, 32.0).astype(jnp.int32)
        iz = jnp.clip(sz, 0.0, 32.0).astype(jnp.int32)
        lin = jnp.minimum(ix * 1024 + iy * 32 + iz, GRID - 1)
        row = lin >> 7
        col = lin & 127
        cx = plsc.load_gather(grid_v, [row, col])
        cy = plsc.load_gather(grid_v, [row + GB, col])
        cz = plsc.load_gather(grid_v, [row + 2 * GB, col])
        dx = sx - cx
        dy = sy - cy
        dz = sz - cz
        ss = dx * dx + dy * dy + dz * dz
        ssc = jnp.maximum(ss, 1e-20)
        bits = lax.bitcast_convert_type(ssc, jnp.int32)
        r = lax.bitcast_convert_type(0x5F3759DF - (bits >> 1), jnp.float32)
        h = 0.5 * ssc
        r = r * (1.5 - h * r * r)
        r = r * (1.5 - h * r * r)
        return acc + ss * r

    def refl_body(j, acc):
        t5 = (j >> 5) * 5
        jr = j & 31
        n0 = cf_v[t5, :]
        n1 = cf_v[t5 + 1, :]
        n2 = cf_v[t5 + 2, :]
        d = cf_v[t5 + 3, :]
        t2 = cf_v[t5 + 4, :]
        for lo in range(0, 128, L):
            x = pts_v[jr, pl.ds(lo, L)]
            y = pts_v[PB + jr, pl.ds(lo, L)]
            z = pts_v[2 * PB + jr, pl.ds(lo, L)]
            s = (n0 * x + n1 * y + n2 * z + d) * t2
            acc = _tail(x - s * n0, y - s * n1, z - s * n2, acc)
        return acc

    def rot_body(j, acc):
        t9 = (j >> 5) * 9
        jr = j & 31
        m = [cf_v[15 + t9 + k, :] for k in range(9)]
        for lo in range(0, 128, L):
            x = pts_v[jr, pl.ds(lo, L)]
            y = pts_v[PB + jr, pl.ds(lo, L)]
            z = pts_v[2 * PB + jr, pl.ds(lo, L)]
            sx = m[0] * x + m[1] * y + m[2] * z
            sy = m[3] * x + m[4] * y + m[5] * z
            sz = m[6] * x + m[7] * y + m[8] * z
            acc = _tail(sx, sy, sz, acc)
        return acc

    acc = jnp.zeros((L,), jnp.float32)
    acc = plsc.parallel_loop(0, 3 * PB, step=1, unroll=2, carry=acc)(refl_body)
    acc = plsc.parallel_loop(0, 3 * PB, step=1, unroll=2, carry=acc)(rot_body)

    out_v[...] = acc
    pltpu.sync_copy(out_v, out_hbm.at[pl.ds(wid * L, L)])


def kernel(output, points, closest):
    grid5 = closest.transpose(2, 0, 1).reshape(3, 2, 8, GB, 128).transpose(0, 1, 3, 2, 4)
    pts5 = points.transpose(2, 0, 1).reshape(3, 2, 8, N // 128, 128).transpose(0, 1, 3, 2, 4)
    prm = output.reshape(-1)

    mesh = plsc.VectorSubcoreMesh(core_axis_name="c", subcore_axis_name="s")
    partials = pl.kernel(
        _sc_body,
        mesh=mesh,
        out_type=jax.ShapeDtypeStruct((NW * L,), jnp.float32),
        scratch_types=[
            pltpu.VMEM((3 * GB, 128), jnp.float32),
            pltpu.VMEM((3 * PB, 128), jnp.float32),
            pltpu.VMEM((B * 24,), jnp.float32),
            pltpu.VMEM((42, L), jnp.float32),
            pltpu.VMEM((L,), jnp.float32),
            pltpu.SemaphoreType.DMA,
        ],
        compiler_params=pltpu.CompilerParams(needs_layout_passes=False),
    )(grid5, pts5, prm)
    return jnp.sum(partials) / B


# speedup vs baseline: 1.2169x; 1.2169x over previous
import jax
import jax.numpy as jnp
from jax import lax
from jax.experimental import pallas as pl
from jax.experimental.pallas import tpu as pltpu
from jax.experimental.pallas import tpu_sc as plsc

B, N, GRID = 16, 8192, 32768
NC, NS, L = 2, 16, 16
NW = NC * NS
HALF = N // 2
GB = GRID // 128
PB = HALF // 128


def _rsqrt(v):
    bits = lax.bitcast_convert_type(v, jnp.int32)
    r = lax.bitcast_convert_type(0x5F3759DF - (bits >> 1), jnp.float32)
    h = 0.5 * v
    r = r * (1.5 - h * r * r)
    r = r * (1.5 - h * r * r)
    r = r * (1.5 - h * r * r)
    return r


def _sc_body(grid_hbm, pts_hbm, prm_hbm, out_hbm, grid_v, pts_v, prm_v, cf_v, out_v, sem):
    wid = lax.axis_index("s") * NC + lax.axis_index("c")
    b = wid // 2
    half = wid % 2
    bhi = b // 8
    blo = b % 8

    cps = [pltpu.make_async_copy(prm_hbm, prm_v, sem)]
    for c in range(3):
        cps.append(pltpu.make_async_copy(
            grid_hbm.at[c, bhi, :, blo, :], grid_v.at[pl.ds(c * GB, GB), :], sem))
        cps.append(pltpu.make_async_copy(
            pts_hbm.at[c, bhi, pl.ds(half * PB, PB), blo, :],
            pts_v.at[pl.ds(c * PB, PB), :], sem))
    for cp in cps:
        cp.start()
    for cp in cps:
        cp.wait()

    for t in range(3):
        base = jnp.full((L,), b * 24 + 4 * t, jnp.int32)
        n0 = plsc.load_gather(prm_v, [base])
        n1 = plsc.load_gather(prm_v, [base + 1])
        n2 = plsc.load_gather(prm_v, [base + 2])
        d = plsc.load_gather(prm_v, [base + 3])
        inv = 1.0 / (n0 * n0 + n1 * n1 + n2 * n2)
        cf_v[5 * t, :] = n0
        cf_v[5 * t + 1, :] = n1
        cf_v[5 * t + 2, :] = n2
        cf_v[5 * t + 3, :] = d
        cf_v[5 * t + 4, :] = inv + inv
    for t in range(3):
        base = jnp.full((L,), b * 24 + 12 + 4 * t, jnp.int32)
        w = plsc.load_gather(prm_v, [base])
        x = plsc.load_gather(prm_v, [base + 1])
        y = plsc.load_gather(prm_v, [base + 2])
        z = plsc.load_gather(prm_v, [base + 3])
        rn = _rsqrt(w * w + x * x + y * y + z * z)
        ww, xx, yy, zz = w * w, x * x, y * y, z * z
        xy, xz, yz = x * y, x * z, y * z
        wx, wy, wz = w * x, w * y, w * z
        m = [
            (ww + xx - yy - zz) * rn, 2.0 * (xy - wz) * rn, 2.0 * (xz + wy) * rn,
            2.0 * (xy + wz) * rn, (ww - xx + yy - zz) * rn, 2.0 * (yz - wx) * rn,
            2.0 * (xz - wy) * rn, 2.0 * (yz + wx) * rn, (ww - xx - yy + zz) * rn,
        ]
        for k in range(9):
            cf_v[15 + 9 * t + k, :] = m[k]

    def _tail(sx, sy, sz, acc):
        ix = jnp.clip(sx, 0.0, 32.0).astype(jnp.int32)
        iy = jnp.clip(sy, 0.0, 32.0).astype(jnp.int32)
        iz = jnp.clip(sz, 0.0, 32.0).astype(jnp.int32)
        lin = jnp.minimum(ix * 1024 + iy * 32 + iz, GRID - 1)
        row = lin >> 7
        col = lin & 127
        cx = plsc.load_gather(grid_v, [row, col])
        cy = plsc.load_gather(grid_v, [row + GB, col])
        cz = plsc.load_gather(grid_v, [row + 2 * GB, col])
        dx = sx - cx
        dy = sy - cy
        dz = sz - cz
        ss = dx * dx + dy * dy + dz * dz
        ssc = jnp.maximum(ss, 1e-20)
        bits = lax.bitcast_convert_type(ssc, jnp.int32)
        r = lax.bitcast_convert_type(0x5F3759DF - (bits >> 1), jnp.float32)
        h = 0.5 * ssc
        r = r * (1.5 - h * r * r)
        r = r * (1.5 - h * r * r)
        return acc + ss * r

    def refl_body(j, acc):
        t5 = (j >> 5) * 5
        jr = j & 31
        n0 = cf_v[t5, :]
        n1 = cf_v[t5 + 1, :]
        n2 = cf_v[t5 + 2, :]
        d = cf_v[t5 + 3, :]
        t2 = cf_v[t5 + 4, :]
        for lo in range(0, 128, L):
            x = pts_v[jr, pl.ds(lo, L)]
            y = pts_v[PB + jr, pl.ds(lo, L)]
            z = pts_v[2 * PB + jr, pl.ds(lo, L)]
            s = (n0 * x + n1 * y + n2 * z + d) * t2
            acc = _tail(x - s * n0, y - s * n1, z - s * n2, acc)
        return acc

    def rot_body(j, acc):
        t9 = (j >> 5) * 9
        jr = j & 31
        m = [cf_v[15 + t9 + k, :] for k in range(9)]
        for lo in range(0, 128, L):
            x = pts_v[jr, pl.ds(lo, L)]
            y = pts_v[PB + jr, pl.ds(lo, L)]
            z = pts_v[2 * PB + jr, pl.ds(lo, L)]
            sx = m[0] * x + m[1] * y + m[2] * z
            sy = m[3] * x + m[4] * y + m[5] * z
            sz = m[6] * x + m[7] * y + m[8] * z
            acc = _tail(sx, sy, sz, acc)
        return acc

    acc = jnp.zeros((L,), jnp.float32)
    acc = plsc.parallel_loop(0, 3 * PB, step=1, unroll=1, carry=acc)(refl_body)
    acc = plsc.parallel_loop(0, 3 * PB, step=1, unroll=1, carry=acc)(rot_body)

    out_v[...] = acc
    pltpu.sync_copy(out_v, out_hbm.at[pl.ds(wid * L, L)])


def kernel(output, points, closest):
    grid5 = closest.transpose(2, 0, 1).reshape(3, 2, 8, GB, 128).transpose(0, 1, 3, 2, 4)
    pts5 = points.transpose(2, 0, 1).reshape(3, 2, 8, N // 128, 128).transpose(0, 1, 3, 2, 4)
    prm = output.reshape(-1)

    mesh = plsc.VectorSubcoreMesh(core_axis_name="c", subcore_axis_name="s")
    partials = pl.kernel(
        _sc_body,
        mesh=mesh,
        out_type=jax.ShapeDtypeStruct((NW * L,), jnp.float32),
        scratch_types=[
            pltpu.VMEM((3 * GB, 128), jnp.float32),
            pltpu.VMEM((3 * PB, 128), jnp.float32),
            pltpu.VMEM((B * 24,), jnp.float32),
            pltpu.VMEM((42, L), jnp.float32),
            pltpu.VMEM((L,), jnp.float32),
            pltpu.SemaphoreType.DMA,
        ],
        compiler_params=pltpu.CompilerParams(needs_layout_passes=False),
    )(grid5, pts5, prm)
    return jnp.sum(partials) / B
